# baseline (speedup 1.0000x reference)
import jax
import jax.numpy as jnp
from jax import lax
from jax.experimental import pallas as pl
from jax.experimental.pallas import tpu as pltpu
from jax.experimental.pallas import tpu_sc as plsc

_NC = 2
_NS = 16
_NW = _NC * _NS
_G = 4


def _sc_add(nt, cw, nch, x_ref, emb_ref, o_ref, emb_v, in_buf, out_buf,
            in_sem0, in_sem1, out_sem0, out_sem1):
    c = lax.axis_index("c")
    s = lax.axis_index("s")
    col0 = (s * _NC + c) * cw
    in_sems = (in_sem0, in_sem1)
    out_sems = (out_sem0, out_sem1)

    pltpu.sync_copy(emb_ref.at[pl.ds(col0, cw)], emb_v)

    def in_copy(i, b):
        return pltpu.make_async_copy(
            x_ref.at[pl.ds(i * _G, _G), pl.ds(col0, cw)],
            in_buf.at[b], in_sems[b])

    def out_copy(i, b):
        return pltpu.make_async_copy(
            out_buf.at[b],
            o_ref.at[pl.ds(i * _G, _G), pl.ds(col0, cw)],
            out_sems[b])

    in_copy(0, 0).start()
    in_copy(1, 1).start()

    def step(i, b):
        in_copy(i, b).wait()

        @pl.when(i >= 2)
        def _():
            out_copy(i - 2, b).wait()

        def jbody(j, carry):
            ds = pl.ds(j * 16, 16)
            e = emb_v[ds]
            for g in range(_G):
                out_buf[b, g, ds] = in_buf[b, g, ds] + e
            return carry

        lax.fori_loop(0, cw // 16, jbody, 0)

        out_copy(i, b).start()

        @pl.when(i + 2 < nch)
        def _():
            in_copy(i + 2, b).start()

    def kbody(k, carry):
        step(k * 2, 0)
        step(k * 2 + 1, 1)
        return carry

    lax.fori_loop(0, nch // 2, kbody, 0)

    out_copy(nch - 2, 0).wait()
    out_copy(nch - 1, 1).wait()


def kernel(x, freqs, emb_weight):
    del freqs
    nt, nf, d = x.shape
    nfd = nf * d
    cw = nfd // _NW
    nch = nt // _G
    assert nfd % _NW == 0 and nt % (2 * _G) == 0 and cw % 16 == 0

    x2 = x.reshape(nt, nfd)
    emb1 = emb_weight.reshape(nfd)

    body = lambda *refs: _sc_add(nt, cw, nch, *refs)
    out2 = pl.kernel(
        body,
        out_type=jax.ShapeDtypeStruct((nt, nfd), x.dtype),
        mesh=plsc.VectorSubcoreMesh(core_axis_name="c", subcore_axis_name="s"),
        scratch_types=[
            pltpu.VMEM((cw,), jnp.float32),
            pltpu.VMEM((2, _G, cw), jnp.float32),
            pltpu.VMEM((2, _G, cw), jnp.float32),
            pltpu.SemaphoreType.DMA,
            pltpu.SemaphoreType.DMA,
            pltpu.SemaphoreType.DMA,
            pltpu.SemaphoreType.DMA,
        ],
    )(x2, emb1)
    return out2.reshape(nt, nf, d)
